# baseline (speedup 1.0000x reference)
import jax
import jax.numpy as jnp
from jax.experimental import pallas as pl
from jax.experimental.pallas import tpu as pltpu

_THRESHOLD = 0.5
_INF = float("inf")
_N = 1024
_STRIP = 256
_NSTRIP = _N // _STRIP


def _shift_cols(a, k, fill):
    if k > 0:
        pad = jnp.full((a.shape[0], k), fill, a.dtype)
        return jnp.concatenate([a[:, k:], pad], axis=1)
    if k < 0:
        pad = jnp.full((a.shape[0], -k), fill, a.dtype)
        return jnp.concatenate([pad, a[:, :k]], axis=1)
    return a


def _erode_cols(x):
    p = jnp.minimum(x, _shift_cols(x, 1, _INF))
    inf_col = jnp.full((x.shape[0], 1), _INF, x.dtype)
    sh = jnp.concatenate([inf_col, x[:, 0:1], p[:, :-2]], axis=1)
    return jnp.minimum(sh, p)


def _dilate_cols(x):
    q = jnp.maximum(_shift_cols(x, -1, -_INF), x)
    ninf_col = jnp.full((x.shape[0], 1), -_INF, x.dtype)
    sh = jnp.concatenate([q[:, 2:], x[:, -1:], ninf_col], axis=1)
    return jnp.maximum(q, sh)


def _mask_kernel(score_ref, mask_ref, score_out_ref, mask_out_ref):
    s = score_ref[0]
    score_out_ref[0] = jnp.where(s >= _THRESHOLD, s, jnp.zeros((), s.dtype))
    s = score_ref[1]
    score_out_ref[1] = jnp.where(s >= _THRESHOLD, s, jnp.zeros((), s.dtype))

    inf_rows8 = jnp.full((8, _N), _INF, jnp.float32)
    ninf_row = jnp.full((1, _N), -_INF, jnp.float32)

    for img in range(2):
      for st in range(_NSTRIP):
          r0 = st * _STRIP
          if st == 0:
              a = jnp.concatenate([inf_rows8, mask_ref[img, 0:_STRIP + 8, :]], axis=0)
          elif st == _NSTRIP - 1:
              a = jnp.concatenate([mask_ref[img, r0 - 8:_N, :], inf_rows8], axis=0)
          else:
              a = mask_ref[img, r0 - 8:r0 + _STRIP + 8, :]

          p = jnp.minimum(a[5:5 + _STRIP + 6, :], a[6:6 + _STRIP + 6, :])
          er = jnp.minimum(p[0:_STRIP + 4, :], p[2:2 + _STRIP + 4, :])

          ec = _erode_cols(er)
          et = jnp.where(ec >= _THRESHOLD, ec, jnp.zeros((), ec.dtype))

          if st == 0:
              et = jnp.concatenate([ninf_row, et[1:, :]], axis=0)
          elif st == _NSTRIP - 1:
              et = jnp.concatenate([et[0:_STRIP + 1, :],
                                    jnp.full((3, _N), -_INF, jnp.float32)], axis=0)

          q = jnp.maximum(et[0:_STRIP + 2, :], et[1:_STRIP + 3, :])
          dr = jnp.maximum(q[0:_STRIP, :], q[2:_STRIP + 2, :])

          mask_out_ref[img, r0:r0 + _STRIP, :] = _dilate_cols(dr)


@jax.jit
def kernel(score, mask):
    b, n = score.shape
    score3 = score.reshape(b, 1, n)
    score_out, mask_out = pl.pallas_call(
        _mask_kernel,
        grid=(b // 2,),
        in_specs=[
            pl.BlockSpec((2, 1, n), lambda i: (i, 0, 0)),
            pl.BlockSpec((2, _N, _N), lambda i: (i, 0, 0)),
        ],
        out_specs=[
            pl.BlockSpec((2, 1, n), lambda i: (i, 0, 0)),
            pl.BlockSpec((2, _N, _N), lambda i: (i, 0, 0)),
        ],
        out_shape=[
            jax.ShapeDtypeStruct(score3.shape, score.dtype),
            jax.ShapeDtypeStruct(mask.shape, mask.dtype),
        ],
        compiler_params=pltpu.CompilerParams(
            dimension_semantics=("arbitrary",),
            vmem_limit_bytes=100 * 1024 * 1024,
        ),
    )(score3, mask)
    return (score_out.reshape(b, n), mask_out)


# speedup vs baseline: 1.0040x; 1.0040x over previous
import jax
import jax.numpy as jnp
from jax.experimental import pallas as pl
from jax.experimental.pallas import tpu as pltpu

_THRESHOLD = 0.5
_INF = float("inf")
_N = 1024
_STRIP = 256
_NSTRIP = _N // _STRIP


def _shift_cols(a, k, fill):
    if k > 0:
        pad = jnp.full((a.shape[0], k), fill, a.dtype)
        return jnp.concatenate([a[:, k:], pad], axis=1)
    if k < 0:
        pad = jnp.full((a.shape[0], -k), fill, a.dtype)
        return jnp.concatenate([pad, a[:, :k]], axis=1)
    return a


def _erode_cols(x):
    p = jnp.minimum(x, _shift_cols(x, 1, _INF))
    inf_col = jnp.full((x.shape[0], 1), _INF, x.dtype)
    sh = jnp.concatenate([inf_col, x[:, 0:1], p[:, :-2]], axis=1)
    return jnp.minimum(sh, p)


def _dilate_cols(x):
    q = jnp.maximum(_shift_cols(x, -1, -_INF), x)
    ninf_col = jnp.full((x.shape[0], 1), -_INF, x.dtype)
    sh = jnp.concatenate([q[:, 2:], x[:, -1:], ninf_col], axis=1)
    return jnp.maximum(q, sh)


def _mask_kernel(score_ref, mask_ref, score_out_ref, mask_out_ref):
    s = score_ref[0]
    score_out_ref[0] = jnp.where(s >= _THRESHOLD, s, jnp.zeros((), s.dtype))

    inf_rows8 = jnp.full((8, _N), _INF, jnp.float32)
    ninf_row = jnp.full((1, _N), -_INF, jnp.float32)

    for st in range(_NSTRIP):
        r0 = st * _STRIP
        if st == 0:
            a = jnp.concatenate([inf_rows8, mask_ref[0, 0:_STRIP + 8, :]], axis=0)
        elif st == _NSTRIP - 1:
            a = jnp.concatenate([mask_ref[0, r0 - 8:_N, :], inf_rows8], axis=0)
        else:
            a = mask_ref[0, r0 - 8:r0 + _STRIP + 8, :]

        p = jnp.minimum(a[5:5 + _STRIP + 6, :], a[6:6 + _STRIP + 6, :])
        er = jnp.minimum(p[0:_STRIP + 4, :], p[2:2 + _STRIP + 4, :])

        ec = _erode_cols(er)
        et = jnp.where(ec >= _THRESHOLD, ec, jnp.zeros((), ec.dtype))

        if st == 0:
            et = jnp.concatenate([ninf_row, et[1:, :]], axis=0)
        elif st == _NSTRIP - 1:
            et = jnp.concatenate([et[0:_STRIP + 1, :],
                                  jnp.full((3, _N), -_INF, jnp.float32)], axis=0)

        q = jnp.maximum(et[0:_STRIP + 2, :], et[1:_STRIP + 3, :])
        dr = jnp.maximum(q[0:_STRIP, :], q[2:_STRIP + 2, :])

        mask_out_ref[0, r0:r0 + _STRIP, :] = _dilate_cols(dr)


@jax.jit
def kernel(score, mask):
    b, n = score.shape
    score3 = score.reshape(b, 1, n)
    score_out, mask_out = pl.pallas_call(
        _mask_kernel,
        grid=(b,),
        in_specs=[
            pl.BlockSpec((1, 1, n), lambda i: (i, 0, 0)),
            pl.BlockSpec((1, _N, _N), lambda i: (i, 0, 0)),
        ],
        out_specs=[
            pl.BlockSpec((1, 1, n), lambda i: (i, 0, 0)),
            pl.BlockSpec((1, _N, _N), lambda i: (i, 0, 0)),
        ],
        out_shape=[
            jax.ShapeDtypeStruct(score3.shape, score.dtype),
            jax.ShapeDtypeStruct(mask.shape, mask.dtype),
        ],
        compiler_params=pltpu.CompilerParams(
            dimension_semantics=("arbitrary",),
            vmem_limit_bytes=100 * 1024 * 1024,
        ),
    )(score3, mask)
    return (score_out.reshape(b, n), mask_out)
